# baseline (speedup 1.0000x reference)
import functools

import jax
import jax.numpy as jnp
from jax import lax
from jax.experimental import pallas as pl
from jax.experimental.pallas import tpu as pltpu
from jax.experimental.pallas import tpu_sc as plsc

BATCH = 16384
EMBED = 32
GROUP = 128 // EMBED
LANES = 16
NUM_CORES = 2
NUM_SUBCORES = 16
NUM_WORKERS = NUM_CORES * NUM_SUBCORES
B_PER_W = BATCH // NUM_WORKERS
CHUNK = 128
NCHUNK = B_PER_W // CHUNK
BLK_PER_CHUNK = CHUNK // LANES


def _tt_body(uid_hbm, mid_hbm, utab_hbm, itab_hbm, out_hbm,
             uidx_v, midx_v, ucol_v, mcol_v,
             ubuf0, ubuf1, ibuf0, ibuf1, out_v, sem0, sem1):
    wid = lax.axis_index("s") * NUM_CORES + lax.axis_index("c")
    base = wid * B_PER_W

    for j in range(NCHUNK):
        pltpu.sync_copy(uid_hbm.at[pl.ds(base + j * CHUNK, CHUNK)], uidx_v.at[j])
        pltpu.sync_copy(mid_hbm.at[pl.ds(base + j * CHUNK, CHUNK)], midx_v.at[j])

    for j in range(NCHUNK):
        for c in range(CHUNK // LANES):
            sl = pl.ds(c * LANES, LANES)
            gsl = pl.ds(j * CHUNK + c * LANES, LANES)
            u = uidx_v[j, sl]
            m = midx_v[j, sl]
            ucol_v[gsl] = (u & (GROUP - 1)) * EMBED
            mcol_v[gsl] = (m & (GROUP - 1)) * EMBED
            uidx_v[j, sl] = jax.lax.shift_right_logical(u, 2)
            midx_v[j, sl] = jax.lax.shift_right_logical(m, 2)

    ubufs = (ubuf0, ubuf1)
    ibufs = (ibuf0, ibuf1)
    sems = (sem0, sem1)

    def fire(k):
        s = sems[k % 2]
        return (pltpu.async_copy(utab_hbm.at[uidx_v.at[k]], ubufs[k % 2], s),
                pltpu.async_copy(itab_hbm.at[midx_v.at[k]], ibufs[k % 2], s))

    lanes = lax.iota(jnp.int32, LANES)
    pend = fire(0)
    for k in range(NCHUNK):
        for h in pend:
            h.wait()
        if k + 1 < NCHUNK:
            pend = fire(k + 1)
        ub = ubufs[k % 2]
        ib = ibufs[k % 2]

        def blk_body(b, carry, ub=ub, ib=ib, k=k):
            rows16 = b * LANES + lanes
            g0 = k * CHUNK
            ucol16 = ucol_v[pl.ds(g0 + b * LANES, LANES)]
            mcol16 = mcol_v[pl.ds(g0 + b * LANES, LANES)]
            acc = jnp.zeros((LANES,), jnp.float32)
            for d in range(EMBED):
                u_d = plsc.load_gather(ub, [rows16, ucol16 + d])
                i_d = plsc.load_gather(ib, [rows16, mcol16 + d])
                acc = acc + u_d * i_d
            out_v[pl.ds(g0 + b * LANES, LANES)] = 1.0 / (1.0 + jnp.exp(-acc))
            return carry

        lax.fori_loop(0, BLK_PER_CHUNK, blk_body, 0)

    pltpu.sync_copy(out_v, out_hbm.at[pl.ds(base, B_PER_W)])


_tt = functools.partial(
    pl.kernel,
    out_type=jax.ShapeDtypeStruct((BATCH,), jnp.float32),
    mesh=plsc.VectorSubcoreMesh(core_axis_name="c", subcore_axis_name="s"),
    scratch_types=[
        pltpu.VMEM((NCHUNK, CHUNK), jnp.int32),
        pltpu.VMEM((NCHUNK, CHUNK), jnp.int32),
        pltpu.VMEM((B_PER_W,), jnp.int32),
        pltpu.VMEM((B_PER_W,), jnp.int32),
        pltpu.VMEM((CHUNK, 128), jnp.float32),
        pltpu.VMEM((CHUNK, 128), jnp.float32),
        pltpu.VMEM((CHUNK, 128), jnp.float32),
        pltpu.VMEM((CHUNK, 128), jnp.float32),
        pltpu.VMEM((B_PER_W,), jnp.float32),
        pltpu.SemaphoreType.DMA,
        pltpu.SemaphoreType.DMA,
    ],
    compiler_params=pltpu.CompilerParams(
        needs_layout_passes=False, use_tc_tiling_on_sc=True),
)(_tt_body)


def kernel(user_id, movie_id, user_table, item_table):
    n_grp = user_table.shape[0] // GROUP
    return _tt(user_id.astype(jnp.int32), movie_id.astype(jnp.int32),
               user_table.reshape(n_grp, GROUP * EMBED),
               item_table.reshape(n_grp, GROUP * EMBED))


# speedup vs baseline: 1.6537x; 1.6537x over previous
import functools

import jax
import jax.numpy as jnp
from jax import lax
from jax.experimental import pallas as pl
from jax.experimental.pallas import tpu as pltpu
from jax.experimental.pallas import tpu_sc as plsc

BATCH = 16384
EMBED = 32
NROWS = 1000000
GROUP = 128 // EMBED
TRW = 2048
NBLK = -(-NROWS // (GROUP * TRW))
SUB = NBLK * TRW
LANES = 16
NUM_CORES = 2
NUM_SUBCORES = 16
NUM_WORKERS = NUM_CORES * NUM_SUBCORES
B_PER_W = BATCH // NUM_WORKERS
CHUNK = 128
NCHUNK = B_PER_W // CHUNK
BLK_PER_CHUNK = CHUNK // LANES


def _tr_body(x0, x1, x2, x3, o_ref):
    eye = (lax.broadcasted_iota(jnp.int32, (EMBED, EMBED), 0)
           == lax.broadcasted_iota(jnp.int32, (EMBED, EMBED), 1)
           ).astype(jnp.float32)
    for s, x in enumerate((x0, x1, x2, x3)):
        xt = lax.dot_general(x[...], eye, (((0,), (0,)), ((), ())),
                             preferred_element_type=jnp.float32)
        o_ref[:, s * EMBED:(s + 1) * EMBED] = xt


_tr = pl.pallas_call(
    _tr_body,
    out_shape=jax.ShapeDtypeStruct((SUB, 128), jnp.float32),
    grid=(NBLK,),
    in_specs=[
        pl.BlockSpec(
            (EMBED, TRW),
            functools.partial(
                lambda i, s: (0, jnp.minimum(s * NBLK + i, -(-NROWS // TRW) - 1)),
                s=s))
        for s in range(GROUP)
    ],
    out_specs=pl.BlockSpec((TRW, 128), lambda i: (i, 0)),
)


def _tt_body(uid_hbm, mid_hbm, utab_hbm, itab_hbm, out_hbm,
             uidx_v, midx_v, ucol_v, mcol_v,
             ubuf0, ubuf1, ibuf0, ibuf1, out_v, sem0, sem1):
    wid = lax.axis_index("s") * NUM_CORES + lax.axis_index("c")
    base = wid * B_PER_W

    for j in range(NCHUNK):
        pltpu.sync_copy(uid_hbm.at[pl.ds(base + j * CHUNK, CHUNK)], uidx_v.at[j])
        pltpu.sync_copy(mid_hbm.at[pl.ds(base + j * CHUNK, CHUNK)], midx_v.at[j])

    for j in range(NCHUNK):
        for c in range(CHUNK // LANES):
            sl = pl.ds(c * LANES, LANES)
            gsl = pl.ds(j * CHUNK + c * LANES, LANES)
            for v_ref, col_ref in ((uidx_v, ucol_v), (midx_v, mcol_v)):
                r = v_ref[j, sl]
                s = ((r >= SUB).astype(jnp.int32)
                     + (r >= 2 * SUB).astype(jnp.int32)
                     + (r >= 3 * SUB).astype(jnp.int32))
                col_ref[gsl] = s * EMBED
                v_ref[j, sl] = r - s * SUB

    ubufs = (ubuf0, ubuf1)
    ibufs = (ibuf0, ibuf1)
    sems = (sem0, sem1)

    def fire(k):
        s = sems[k % 2]
        return (pltpu.async_copy(utab_hbm.at[uidx_v.at[k]], ubufs[k % 2], s),
                pltpu.async_copy(itab_hbm.at[midx_v.at[k]], ibufs[k % 2], s))

    lanes = lax.iota(jnp.int32, LANES)
    pend = fire(0)
    for k in range(NCHUNK):
        for h in pend:
            h.wait()
        if k + 1 < NCHUNK:
            pend = fire(k + 1)
        ub = ubufs[k % 2]
        ib = ibufs[k % 2]

        def blk_body(b, carry, ub=ub, ib=ib, k=k):
            rows16 = b * LANES + lanes
            g0 = k * CHUNK
            ucol16 = ucol_v[pl.ds(g0 + b * LANES, LANES)]
            mcol16 = mcol_v[pl.ds(g0 + b * LANES, LANES)]
            acc = jnp.zeros((LANES,), jnp.float32)
            for d in range(EMBED):
                u_d = plsc.load_gather(ub, [rows16, ucol16 + d])
                i_d = plsc.load_gather(ib, [rows16, mcol16 + d])
                acc = acc + u_d * i_d
            out_v[pl.ds(g0 + b * LANES, LANES)] = 1.0 / (1.0 + jnp.exp(-acc))
            return carry

        lax.fori_loop(0, BLK_PER_CHUNK, blk_body, 0)

    pltpu.sync_copy(out_v, out_hbm.at[pl.ds(base, B_PER_W)])


_tt = functools.partial(
    pl.kernel,
    out_type=jax.ShapeDtypeStruct((BATCH,), jnp.float32),
    mesh=plsc.VectorSubcoreMesh(core_axis_name="c", subcore_axis_name="s"),
    scratch_types=[
        pltpu.VMEM((NCHUNK, CHUNK), jnp.int32),
        pltpu.VMEM((NCHUNK, CHUNK), jnp.int32),
        pltpu.VMEM((B_PER_W,), jnp.int32),
        pltpu.VMEM((B_PER_W,), jnp.int32),
        pltpu.VMEM((CHUNK, 128), jnp.float32),
        pltpu.VMEM((CHUNK, 128), jnp.float32),
        pltpu.VMEM((CHUNK, 128), jnp.float32),
        pltpu.VMEM((CHUNK, 128), jnp.float32),
        pltpu.VMEM((B_PER_W,), jnp.float32),
        pltpu.SemaphoreType.DMA,
        pltpu.SemaphoreType.DMA,
    ],
    compiler_params=pltpu.CompilerParams(
        needs_layout_passes=False, use_tc_tiling_on_sc=True),
)(_tt_body)


def kernel(user_id, movie_id, user_table, item_table):
    utt = user_table.T
    itt = item_table.T
    ut4 = _tr(utt, utt, utt, utt)
    it4 = _tr(itt, itt, itt, itt)
    return _tt(user_id.astype(jnp.int32), movie_id.astype(jnp.int32),
               ut4, it4)


# speedup vs baseline: 2.8843x; 1.7441x over previous
import functools

import jax
import jax.numpy as jnp
from jax import lax
from jax.experimental import pallas as pl
from jax.experimental.pallas import tpu as pltpu
from jax.experimental.pallas import tpu_sc as plsc

BATCH = 16384
EMBED = 32
NROWS = 1000000
GROUP = 128 // EMBED
TRW = 2048
NBLK = -(-NROWS // (GROUP * TRW))
SUB = NBLK * TRW
LANES = 16
NUM_CORES = 2
NUM_SUBCORES = 16
NUM_WORKERS = NUM_CORES * NUM_SUBCORES
B_PER_W = BATCH // NUM_WORKERS
CHUNK = 128
NCHUNK = B_PER_W // CHUNK
BLK_PER_CHUNK = CHUNK // LANES


def _tr_body(x0, x1, x2, x3, o_ref):
    eye = (lax.broadcasted_iota(jnp.int32, (128, 128), 0)
           == lax.broadcasted_iota(jnp.int32, (128, 128), 1)
           ).astype(jnp.float32)
    xcat = jnp.concatenate([x0[...], x1[...], x2[...], x3[...]], axis=0)
    o_ref[...] = lax.dot_general(xcat, eye, (((0,), (0,)), ((), ())),
                                 preferred_element_type=jnp.float32)


_tr = pl.pallas_call(
    _tr_body,
    out_shape=jax.ShapeDtypeStruct((SUB, 128), jnp.float32),
    grid=(NBLK,),
    in_specs=[
        pl.BlockSpec(
            (EMBED, TRW),
            functools.partial(
                lambda i, s: (0, jnp.minimum(s * NBLK + i, -(-NROWS // TRW) - 1)),
                s=s))
        for s in range(GROUP)
    ],
    out_specs=pl.BlockSpec((TRW, 128), lambda i: (i, 0)),
)


def _tt_body(uid_hbm, mid_hbm, utab_hbm, itab_hbm, out_hbm,
             uidx_v, midx_v, ucol_v, mcol_v,
             ubuf0, ubuf1, ibuf0, ibuf1, out_v, sem0, sem1):
    wid = lax.axis_index("s") * NUM_CORES + lax.axis_index("c")
    base = wid * B_PER_W

    for j in range(NCHUNK):
        pltpu.sync_copy(uid_hbm.at[pl.ds(base + j * CHUNK, CHUNK)], uidx_v.at[j])
        pltpu.sync_copy(mid_hbm.at[pl.ds(base + j * CHUNK, CHUNK)], midx_v.at[j])

    for j in range(NCHUNK):
        for c in range(CHUNK // LANES):
            sl = pl.ds(c * LANES, LANES)
            gsl = pl.ds(j * CHUNK + c * LANES, LANES)
            for v_ref, col_ref in ((uidx_v, ucol_v), (midx_v, mcol_v)):
                r = v_ref[j, sl]
                s = ((r >= SUB).astype(jnp.int32)
                     + (r >= 2 * SUB).astype(jnp.int32)
                     + (r >= 3 * SUB).astype(jnp.int32))
                col_ref[gsl] = s * EMBED
                v_ref[j, sl] = r - s * SUB

    ubufs = (ubuf0, ubuf1)
    ibufs = (ibuf0, ibuf1)
    sems = (sem0, sem1)

    def fire(k):
        s = sems[k % 2]
        return (pltpu.async_copy(utab_hbm.at[uidx_v.at[k]], ubufs[k % 2], s),
                pltpu.async_copy(itab_hbm.at[midx_v.at[k]], ibufs[k % 2], s))

    lanes = lax.iota(jnp.int32, LANES)
    pend = fire(0)
    for k in range(NCHUNK):
        for h in pend:
            h.wait()
        if k + 1 < NCHUNK:
            pend = fire(k + 1)
        ub = ubufs[k % 2]
        ib = ibufs[k % 2]

        def blk_body(b, carry, ub=ub, ib=ib, k=k):
            rows16 = b * LANES + lanes
            g0 = k * CHUNK
            ucol16 = ucol_v[pl.ds(g0 + b * LANES, LANES)]
            mcol16 = mcol_v[pl.ds(g0 + b * LANES, LANES)]
            acc = jnp.zeros((LANES,), jnp.float32)
            for d in range(EMBED):
                u_d = plsc.load_gather(ub, [rows16, ucol16 + d])
                i_d = plsc.load_gather(ib, [rows16, mcol16 + d])
                acc = acc + u_d * i_d
            out_v[pl.ds(g0 + b * LANES, LANES)] = 1.0 / (1.0 + jnp.exp(-acc))
            return carry

        lax.fori_loop(0, BLK_PER_CHUNK, blk_body, 0)

    pltpu.sync_copy(out_v, out_hbm.at[pl.ds(base, B_PER_W)])


_tt = functools.partial(
    pl.kernel,
    out_type=jax.ShapeDtypeStruct((BATCH,), jnp.float32),
    mesh=plsc.VectorSubcoreMesh(core_axis_name="c", subcore_axis_name="s"),
    scratch_types=[
        pltpu.VMEM((NCHUNK, CHUNK), jnp.int32),
        pltpu.VMEM((NCHUNK, CHUNK), jnp.int32),
        pltpu.VMEM((B_PER_W,), jnp.int32),
        pltpu.VMEM((B_PER_W,), jnp.int32),
        pltpu.VMEM((CHUNK, 128), jnp.float32),
        pltpu.VMEM((CHUNK, 128), jnp.float32),
        pltpu.VMEM((CHUNK, 128), jnp.float32),
        pltpu.VMEM((CHUNK, 128), jnp.float32),
        pltpu.VMEM((B_PER_W,), jnp.float32),
        pltpu.SemaphoreType.DMA,
        pltpu.SemaphoreType.DMA,
    ],
    compiler_params=pltpu.CompilerParams(
        needs_layout_passes=False, use_tc_tiling_on_sc=True),
)(_tt_body)


def kernel(user_id, movie_id, user_table, item_table):
    utt = user_table.T
    itt = item_table.T
    ut4 = _tr(utt, utt, utt, utt)
    it4 = _tr(itt, itt, itt, itt)
    return _tt(user_id.astype(jnp.int32), movie_id.astype(jnp.int32),
               ut4, it4)


# speedup vs baseline: 3.8240x; 1.3258x over previous
import functools

import jax
import jax.numpy as jnp
from jax import lax
from jax.experimental import pallas as pl
from jax.experimental.pallas import tpu as pltpu
from jax.experimental.pallas import tpu_sc as plsc

BATCH = 16384
EMBED = 32
NROWS = 1000000
GROUP = 128 // EMBED
TRW = 4096
NBLK = -(-NROWS // (GROUP * TRW))
SUB = NBLK * TRW
LANES = 16
NUM_CORES = 2
NUM_SUBCORES = 16
NUM_WORKERS = NUM_CORES * NUM_SUBCORES
B_PER_W = BATCH // NUM_WORKERS
CHUNK = 128
NCHUNK = B_PER_W // CHUNK
BLK_PER_CHUNK = CHUNK // LANES


def _tr_body(x0, x1, x2, x3, o_ref):
    eye = (lax.broadcasted_iota(jnp.int32, (128, 128), 0)
           == lax.broadcasted_iota(jnp.int32, (128, 128), 1)
           ).astype(jnp.float32)
    xcat = jnp.concatenate([x0[...], x1[...], x2[...], x3[...]], axis=0)
    o_ref[...] = lax.dot_general(xcat, eye, (((0,), (0,)), ((), ())),
                                 preferred_element_type=jnp.float32)


_tr = pl.pallas_call(
    _tr_body,
    out_shape=jax.ShapeDtypeStruct((SUB, 128), jnp.float32),
    grid=(NBLK,),
    in_specs=[
        pl.BlockSpec(
            (EMBED, TRW),
            functools.partial(
                lambda i, s: (0, jnp.minimum(s * NBLK + i, -(-NROWS // TRW) - 1)),
                s=s))
        for s in range(GROUP)
    ],
    out_specs=pl.BlockSpec((TRW, 128), lambda i: (i, 0)),
)


def _tt_body(uid_hbm, mid_hbm, utab_hbm, itab_hbm, out_hbm,
             uidx_v, midx_v, ucol_v, mcol_v,
             ubuf0, ubuf1, ibuf0, ibuf1, out_v, sem0, sem1):
    wid = lax.axis_index("s") * NUM_CORES + lax.axis_index("c")
    base = wid * B_PER_W

    for j in range(NCHUNK):
        pltpu.sync_copy(uid_hbm.at[pl.ds(base + j * CHUNK, CHUNK)], uidx_v.at[j])
        pltpu.sync_copy(mid_hbm.at[pl.ds(base + j * CHUNK, CHUNK)], midx_v.at[j])

    for j in range(NCHUNK):
        for c in range(CHUNK // LANES):
            sl = pl.ds(c * LANES, LANES)
            gsl = pl.ds(j * CHUNK + c * LANES, LANES)
            for v_ref, col_ref in ((uidx_v, ucol_v), (midx_v, mcol_v)):
                r = v_ref[j, sl]
                s = ((r >= SUB).astype(jnp.int32)
                     + (r >= 2 * SUB).astype(jnp.int32)
                     + (r >= 3 * SUB).astype(jnp.int32))
                col_ref[gsl] = s * EMBED
                v_ref[j, sl] = r - s * SUB

    ubufs = (ubuf0, ubuf1)
    ibufs = (ibuf0, ibuf1)
    sems = (sem0, sem1)

    def fire(k):
        s = sems[k % 2]
        return (pltpu.async_copy(utab_hbm.at[uidx_v.at[k]], ubufs[k % 2], s),
                pltpu.async_copy(itab_hbm.at[midx_v.at[k]], ibufs[k % 2], s))

    lanes = lax.iota(jnp.int32, LANES)
    pend = fire(0)
    for k in range(NCHUNK):
        for h in pend:
            h.wait()
        if k + 1 < NCHUNK:
            pend = fire(k + 1)
        ub = ubufs[k % 2]
        ib = ibufs[k % 2]

        def blk_body(b, carry, ub=ub, ib=ib, k=k):
            rows16 = b * LANES + lanes
            g0 = k * CHUNK
            ucol16 = ucol_v[pl.ds(g0 + b * LANES, LANES)]
            mcol16 = mcol_v[pl.ds(g0 + b * LANES, LANES)]
            acc = jnp.zeros((LANES,), jnp.float32)
            for d in range(EMBED):
                u_d = plsc.load_gather(ub, [rows16, ucol16 + d])
                i_d = plsc.load_gather(ib, [rows16, mcol16 + d])
                acc = acc + u_d * i_d
            out_v[pl.ds(g0 + b * LANES, LANES)] = 1.0 / (1.0 + jnp.exp(-acc))
            return carry

        lax.fori_loop(0, BLK_PER_CHUNK, blk_body, 0)

    pltpu.sync_copy(out_v, out_hbm.at[pl.ds(base, B_PER_W)])


_tt = functools.partial(
    pl.kernel,
    out_type=jax.ShapeDtypeStruct((BATCH,), jnp.float32),
    mesh=plsc.VectorSubcoreMesh(core_axis_name="c", subcore_axis_name="s"),
    scratch_types=[
        pltpu.VMEM((NCHUNK, CHUNK), jnp.int32),
        pltpu.VMEM((NCHUNK, CHUNK), jnp.int32),
        pltpu.VMEM((B_PER_W,), jnp.int32),
        pltpu.VMEM((B_PER_W,), jnp.int32),
        pltpu.VMEM((CHUNK, 128), jnp.float32),
        pltpu.VMEM((CHUNK, 128), jnp.float32),
        pltpu.VMEM((CHUNK, 128), jnp.float32),
        pltpu.VMEM((CHUNK, 128), jnp.float32),
        pltpu.VMEM((B_PER_W,), jnp.float32),
        pltpu.SemaphoreType.DMA,
        pltpu.SemaphoreType.DMA,
    ],
    compiler_params=pltpu.CompilerParams(
        needs_layout_passes=False, use_tc_tiling_on_sc=True),
)(_tt_body)


def kernel(user_id, movie_id, user_table, item_table):
    utt = user_table.T
    itt = item_table.T
    ut4 = _tr(utt, utt, utt, utt)
    it4 = _tr(itt, itt, itt, itt)
    return _tt(user_id.astype(jnp.int32), movie_id.astype(jnp.int32),
               ut4, it4)


# speedup vs baseline: 4.3620x; 1.1407x over previous
import functools

import jax
import jax.numpy as jnp
from jax import lax
from jax.experimental import pallas as pl
from jax.experimental.pallas import tpu as pltpu
from jax.experimental.pallas import tpu_sc as plsc

BATCH = 16384
EMBED = 32
NROWS = 1000000
GROUP = 128 // EMBED
TRW = 8192
NBLK = -(-NROWS // (GROUP * TRW))
SUB = NBLK * TRW
LANES = 16
NUM_CORES = 2
NUM_SUBCORES = 16
NUM_WORKERS = NUM_CORES * NUM_SUBCORES
B_PER_W = BATCH // NUM_WORKERS
CHUNK = 128
NCHUNK = B_PER_W // CHUNK
BLK_PER_CHUNK = CHUNK // LANES


def _tr_body(x0, x1, x2, x3, o_ref):
    eye = (lax.broadcasted_iota(jnp.int32, (128, 128), 0)
           == lax.broadcasted_iota(jnp.int32, (128, 128), 1)
           ).astype(jnp.float32)
    xcat = jnp.concatenate([x0[...], x1[...], x2[...], x3[...]], axis=0)
    o_ref[...] = lax.dot_general(xcat, eye, (((0,), (0,)), ((), ())),
                                 preferred_element_type=jnp.float32)


_tr = pl.pallas_call(
    _tr_body,
    out_shape=jax.ShapeDtypeStruct((SUB, 128), jnp.float32),
    grid=(NBLK,),
    in_specs=[
        pl.BlockSpec(
            (EMBED, TRW),
            functools.partial(
                lambda i, s: (0, jnp.minimum(s * NBLK + i, -(-NROWS // TRW) - 1)),
                s=s))
        for s in range(GROUP)
    ],
    out_specs=pl.BlockSpec((TRW, 128), lambda i: (i, 0)),
)


def _tt_body(uid_hbm, mid_hbm, utab_hbm, itab_hbm, out_hbm,
             uidx_v, midx_v, ucol_v, mcol_v,
             ubuf0, ubuf1, ibuf0, ibuf1, out_v, sem0, sem1):
    wid = lax.axis_index("s") * NUM_CORES + lax.axis_index("c")
    base = wid * B_PER_W

    for j in range(NCHUNK):
        pltpu.sync_copy(uid_hbm.at[pl.ds(base + j * CHUNK, CHUNK)], uidx_v.at[j])
        pltpu.sync_copy(mid_hbm.at[pl.ds(base + j * CHUNK, CHUNK)], midx_v.at[j])

    for j in range(NCHUNK):
        for c in range(CHUNK // LANES):
            sl = pl.ds(c * LANES, LANES)
            gsl = pl.ds(j * CHUNK + c * LANES, LANES)
            for v_ref, col_ref in ((uidx_v, ucol_v), (midx_v, mcol_v)):
                r = v_ref[j, sl]
                s = ((r >= SUB).astype(jnp.int32)
                     + (r >= 2 * SUB).astype(jnp.int32)
                     + (r >= 3 * SUB).astype(jnp.int32))
                col_ref[gsl] = s * EMBED
                v_ref[j, sl] = r - s * SUB

    ubufs = (ubuf0, ubuf1)
    ibufs = (ibuf0, ibuf1)
    sems = (sem0, sem1)

    def fire(k):
        s = sems[k % 2]
        return (pltpu.async_copy(utab_hbm.at[uidx_v.at[k]], ubufs[k % 2], s),
                pltpu.async_copy(itab_hbm.at[midx_v.at[k]], ibufs[k % 2], s))

    lanes = lax.iota(jnp.int32, LANES)
    pend = fire(0)
    for k in range(NCHUNK):
        for h in pend:
            h.wait()
        if k + 1 < NCHUNK:
            pend = fire(k + 1)
        ub = ubufs[k % 2]
        ib = ibufs[k % 2]

        def blk_body(b, carry, ub=ub, ib=ib, k=k):
            rows16 = b * LANES + lanes
            g0 = k * CHUNK
            ucol16 = ucol_v[pl.ds(g0 + b * LANES, LANES)]
            mcol16 = mcol_v[pl.ds(g0 + b * LANES, LANES)]
            acc = jnp.zeros((LANES,), jnp.float32)
            for d in range(EMBED):
                u_d = plsc.load_gather(ub, [rows16, ucol16 + d])
                i_d = plsc.load_gather(ib, [rows16, mcol16 + d])
                acc = acc + u_d * i_d
            out_v[pl.ds(g0 + b * LANES, LANES)] = 1.0 / (1.0 + jnp.exp(-acc))
            return carry

        lax.fori_loop(0, BLK_PER_CHUNK, blk_body, 0)

    pltpu.sync_copy(out_v, out_hbm.at[pl.ds(base, B_PER_W)])


_tt = functools.partial(
    pl.kernel,
    out_type=jax.ShapeDtypeStruct((BATCH,), jnp.float32),
    mesh=plsc.VectorSubcoreMesh(core_axis_name="c", subcore_axis_name="s"),
    scratch_types=[
        pltpu.VMEM((NCHUNK, CHUNK), jnp.int32),
        pltpu.VMEM((NCHUNK, CHUNK), jnp.int32),
        pltpu.VMEM((B_PER_W,), jnp.int32),
        pltpu.VMEM((B_PER_W,), jnp.int32),
        pltpu.VMEM((CHUNK, 128), jnp.float32),
        pltpu.VMEM((CHUNK, 128), jnp.float32),
        pltpu.VMEM((CHUNK, 128), jnp.float32),
        pltpu.VMEM((CHUNK, 128), jnp.float32),
        pltpu.VMEM((B_PER_W,), jnp.float32),
        pltpu.SemaphoreType.DMA,
        pltpu.SemaphoreType.DMA,
    ],
    compiler_params=pltpu.CompilerParams(
        needs_layout_passes=False, use_tc_tiling_on_sc=True),
)(_tt_body)


def kernel(user_id, movie_id, user_table, item_table):
    utt = user_table.T
    itt = item_table.T
    ut4 = _tr(utt, utt, utt, utt)
    it4 = _tr(itt, itt, itt, itt)
    return _tt(user_id.astype(jnp.int32), movie_id.astype(jnp.int32),
               ut4, it4)


# speedup vs baseline: 4.3853x; 1.0054x over previous
import functools

import jax
import jax.numpy as jnp
from jax import lax
from jax.experimental import pallas as pl
from jax.experimental.pallas import tpu as pltpu
from jax.experimental.pallas import tpu_sc as plsc

BATCH = 16384
EMBED = 32
NROWS = 1000000
GROUP = 128 // EMBED
TRW = 16384
NBLK = -(-NROWS // (GROUP * TRW))
SUB = NBLK * TRW
LANES = 16
NUM_CORES = 2
NUM_SUBCORES = 16
NUM_WORKERS = NUM_CORES * NUM_SUBCORES
B_PER_W = BATCH // NUM_WORKERS
CHUNK = 128
NCHUNK = B_PER_W // CHUNK
BLK_PER_CHUNK = CHUNK // LANES


def _tr_body(x0, x1, x2, x3, o_ref):
    eye = (lax.broadcasted_iota(jnp.int32, (128, 128), 0)
           == lax.broadcasted_iota(jnp.int32, (128, 128), 1)
           ).astype(jnp.float32)
    xcat = jnp.concatenate([x0[...], x1[...], x2[...], x3[...]], axis=0)
    o_ref[...] = lax.dot_general(xcat, eye, (((0,), (0,)), ((), ())),
                                 preferred_element_type=jnp.float32)


_tr = pl.pallas_call(
    _tr_body,
    out_shape=jax.ShapeDtypeStruct((SUB, 128), jnp.float32),
    grid=(NBLK,),
    in_specs=[
        pl.BlockSpec(
            (EMBED, TRW),
            functools.partial(
                lambda i, s: (0, jnp.minimum(s * NBLK + i, -(-NROWS // TRW) - 1)),
                s=s))
        for s in range(GROUP)
    ],
    out_specs=pl.BlockSpec((TRW, 128), lambda i: (i, 0)),
)


def _tt_body(uid_hbm, mid_hbm, utab_hbm, itab_hbm, out_hbm,
             uidx_v, midx_v, ucol_v, mcol_v,
             ubuf0, ubuf1, ibuf0, ibuf1, out_v, sem0, sem1):
    wid = lax.axis_index("s") * NUM_CORES + lax.axis_index("c")
    base = wid * B_PER_W

    for j in range(NCHUNK):
        pltpu.sync_copy(uid_hbm.at[pl.ds(base + j * CHUNK, CHUNK)], uidx_v.at[j])
        pltpu.sync_copy(mid_hbm.at[pl.ds(base + j * CHUNK, CHUNK)], midx_v.at[j])

    for j in range(NCHUNK):
        for c in range(CHUNK // LANES):
            sl = pl.ds(c * LANES, LANES)
            gsl = pl.ds(j * CHUNK + c * LANES, LANES)
            for v_ref, col_ref in ((uidx_v, ucol_v), (midx_v, mcol_v)):
                r = v_ref[j, sl]
                s = ((r >= SUB).astype(jnp.int32)
                     + (r >= 2 * SUB).astype(jnp.int32)
                     + (r >= 3 * SUB).astype(jnp.int32))
                col_ref[gsl] = s * EMBED
                v_ref[j, sl] = r - s * SUB

    ubufs = (ubuf0, ubuf1)
    ibufs = (ibuf0, ibuf1)
    sems = (sem0, sem1)

    def fire(k):
        s = sems[k % 2]
        return (pltpu.async_copy(utab_hbm.at[uidx_v.at[k]], ubufs[k % 2], s),
                pltpu.async_copy(itab_hbm.at[midx_v.at[k]], ibufs[k % 2], s))

    lanes = lax.iota(jnp.int32, LANES)
    pend = fire(0)
    for k in range(NCHUNK):
        for h in pend:
            h.wait()
        if k + 1 < NCHUNK:
            pend = fire(k + 1)
        ub = ubufs[k % 2]
        ib = ibufs[k % 2]

        def blk_body(b, carry, ub=ub, ib=ib, k=k):
            rows16 = b * LANES + lanes
            g0 = k * CHUNK
            ucol16 = ucol_v[pl.ds(g0 + b * LANES, LANES)]
            mcol16 = mcol_v[pl.ds(g0 + b * LANES, LANES)]
            acc = jnp.zeros((LANES,), jnp.float32)
            for d in range(EMBED):
                u_d = plsc.load_gather(ub, [rows16, ucol16 + d])
                i_d = plsc.load_gather(ib, [rows16, mcol16 + d])
                acc = acc + u_d * i_d
            out_v[pl.ds(g0 + b * LANES, LANES)] = 1.0 / (1.0 + jnp.exp(-acc))
            return carry

        lax.fori_loop(0, BLK_PER_CHUNK, blk_body, 0)

    pltpu.sync_copy(out_v, out_hbm.at[pl.ds(base, B_PER_W)])


_tt = functools.partial(
    pl.kernel,
    out_type=jax.ShapeDtypeStruct((BATCH,), jnp.float32),
    mesh=plsc.VectorSubcoreMesh(core_axis_name="c", subcore_axis_name="s"),
    scratch_types=[
        pltpu.VMEM((NCHUNK, CHUNK), jnp.int32),
        pltpu.VMEM((NCHUNK, CHUNK), jnp.int32),
        pltpu.VMEM((B_PER_W,), jnp.int32),
        pltpu.VMEM((B_PER_W,), jnp.int32),
        pltpu.VMEM((CHUNK, 128), jnp.float32),
        pltpu.VMEM((CHUNK, 128), jnp.float32),
        pltpu.VMEM((CHUNK, 128), jnp.float32),
        pltpu.VMEM((CHUNK, 128), jnp.float32),
        pltpu.VMEM((B_PER_W,), jnp.float32),
        pltpu.SemaphoreType.DMA,
        pltpu.SemaphoreType.DMA,
    ],
    compiler_params=pltpu.CompilerParams(
        needs_layout_passes=False, use_tc_tiling_on_sc=True),
)(_tt_body)


def kernel(user_id, movie_id, user_table, item_table):
    utt = user_table.T
    itt = item_table.T
    ut4 = _tr(utt, utt, utt, utt)
    it4 = _tr(itt, itt, itt, itt)
    return _tt(user_id.astype(jnp.int32), movie_id.astype(jnp.int32),
               ut4, it4)


# speedup vs baseline: 4.5101x; 1.0284x over previous
import functools

import jax
import jax.numpy as jnp
from jax import lax
from jax.experimental import pallas as pl
from jax.experimental.pallas import tpu as pltpu
from jax.experimental.pallas import tpu_sc as plsc

BATCH = 16384
EMBED = 32
NROWS = 1000000
GROUP = 128 // EMBED
TRW = 8192
NBLK = -(-NROWS // (GROUP * TRW))
SUB = NBLK * TRW
LANES = 16
NUM_CORES = 2
NUM_SUBCORES = 16
NUM_WORKERS = NUM_CORES * NUM_SUBCORES
B_PER_W = BATCH // NUM_WORKERS
CHUNK = 128
NCHUNK = B_PER_W // CHUNK
BLK_PER_CHUNK = CHUNK // LANES


def _tr_body(u0, u1, u2, u3, i0, i1, i2, i3, ou_ref, oi_ref):
    eye = (lax.broadcasted_iota(jnp.int32, (128, 128), 0)
           == lax.broadcasted_iota(jnp.int32, (128, 128), 1)
           ).astype(jnp.float32)
    ucat = jnp.concatenate([u0[...], u1[...], u2[...], u3[...]], axis=0)
    ou_ref[...] = lax.dot_general(ucat, eye, (((0,), (0,)), ((), ())),
                                  preferred_element_type=jnp.float32)
    icat = jnp.concatenate([i0[...], i1[...], i2[...], i3[...]], axis=0)
    oi_ref[...] = lax.dot_general(icat, eye, (((0,), (0,)), ((), ())),
                                  preferred_element_type=jnp.float32)


_in_specs = [
    pl.BlockSpec(
        (EMBED, TRW),
        functools.partial(
            lambda i, s: (0, jnp.minimum(s * NBLK + i, -(-NROWS // TRW) - 1)),
            s=s))
    for s in range(GROUP)
]

_tr = pl.pallas_call(
    _tr_body,
    out_shape=(jax.ShapeDtypeStruct((SUB, 128), jnp.float32),
               jax.ShapeDtypeStruct((SUB, 128), jnp.float32)),
    grid=(NBLK,),
    in_specs=_in_specs + _in_specs,
    out_specs=(pl.BlockSpec((TRW, 128), lambda i: (i, 0)),
               pl.BlockSpec((TRW, 128), lambda i: (i, 0))),
)


def _tt_body(uid_hbm, mid_hbm, utab_hbm, itab_hbm, out_hbm,
             uidx_v, midx_v, ucol_v, mcol_v,
             ubuf0, ubuf1, ibuf0, ibuf1, out_v, sem0, sem1):
    wid = lax.axis_index("s") * NUM_CORES + lax.axis_index("c")
    base = wid * B_PER_W

    for j in range(NCHUNK):
        pltpu.sync_copy(uid_hbm.at[pl.ds(base + j * CHUNK, CHUNK)], uidx_v.at[j])
        pltpu.sync_copy(mid_hbm.at[pl.ds(base + j * CHUNK, CHUNK)], midx_v.at[j])

    for j in range(NCHUNK):
        for c in range(CHUNK // LANES):
            sl = pl.ds(c * LANES, LANES)
            gsl = pl.ds(j * CHUNK + c * LANES, LANES)
            for v_ref, col_ref in ((uidx_v, ucol_v), (midx_v, mcol_v)):
                r = v_ref[j, sl]
                s = ((r >= SUB).astype(jnp.int32)
                     + (r >= 2 * SUB).astype(jnp.int32)
                     + (r >= 3 * SUB).astype(jnp.int32))
                col_ref[gsl] = s * EMBED
                v_ref[j, sl] = r - s * SUB

    ubufs = (ubuf0, ubuf1)
    ibufs = (ibuf0, ibuf1)
    sems = (sem0, sem1)

    def fire(k):
        s = sems[k % 2]
        return (pltpu.async_copy(utab_hbm.at[uidx_v.at[k]], ubufs[k % 2], s),
                pltpu.async_copy(itab_hbm.at[midx_v.at[k]], ibufs[k % 2], s))

    lanes = lax.iota(jnp.int32, LANES)
    pend = fire(0)
    for k in range(NCHUNK):
        for h in pend:
            h.wait()
        if k + 1 < NCHUNK:
            pend = fire(k + 1)
        ub = ubufs[k % 2]
        ib = ibufs[k % 2]

        def blk_body(b, carry, ub=ub, ib=ib, k=k):
            rows16 = b * LANES + lanes
            g0 = k * CHUNK
            ucol16 = ucol_v[pl.ds(g0 + b * LANES, LANES)]
            mcol16 = mcol_v[pl.ds(g0 + b * LANES, LANES)]
            acc = jnp.zeros((LANES,), jnp.float32)
            for d in range(EMBED):
                u_d = plsc.load_gather(ub, [rows16, ucol16 + d])
                i_d = plsc.load_gather(ib, [rows16, mcol16 + d])
                acc = acc + u_d * i_d
            out_v[pl.ds(g0 + b * LANES, LANES)] = 1.0 / (1.0 + jnp.exp(-acc))
            return carry

        lax.fori_loop(0, BLK_PER_CHUNK, blk_body, 0)

    pltpu.sync_copy(out_v, out_hbm.at[pl.ds(base, B_PER_W)])


_tt = functools.partial(
    pl.kernel,
    out_type=jax.ShapeDtypeStruct((BATCH,), jnp.float32),
    mesh=plsc.VectorSubcoreMesh(core_axis_name="c", subcore_axis_name="s"),
    scratch_types=[
        pltpu.VMEM((NCHUNK, CHUNK), jnp.int32),
        pltpu.VMEM((NCHUNK, CHUNK), jnp.int32),
        pltpu.VMEM((B_PER_W,), jnp.int32),
        pltpu.VMEM((B_PER_W,), jnp.int32),
        pltpu.VMEM((CHUNK, 128), jnp.float32),
        pltpu.VMEM((CHUNK, 128), jnp.float32),
        pltpu.VMEM((CHUNK, 128), jnp.float32),
        pltpu.VMEM((CHUNK, 128), jnp.float32),
        pltpu.VMEM((B_PER_W,), jnp.float32),
        pltpu.SemaphoreType.DMA,
        pltpu.SemaphoreType.DMA,
    ],
    compiler_params=pltpu.CompilerParams(
        needs_layout_passes=False, use_tc_tiling_on_sc=True),
)(_tt_body)


def kernel(user_id, movie_id, user_table, item_table):
    utt = user_table.T
    itt = item_table.T
    ut4, it4 = _tr(utt, utt, utt, utt, itt, itt, itt, itt)
    return _tt(user_id.astype(jnp.int32), movie_id.astype(jnp.int32),
               ut4, it4)


# speedup vs baseline: 4.5114x; 1.0003x over previous
import functools

import jax
import jax.numpy as jnp
from jax import lax
from jax.experimental import pallas as pl
from jax.experimental.pallas import tpu as pltpu
from jax.experimental.pallas import tpu_sc as plsc

BATCH = 16384
EMBED = 32
NROWS = 1000000
GROUP = 128 // EMBED
TRW = 12288
NBLK = -(-NROWS // (GROUP * TRW))
SUB = NBLK * TRW
LANES = 16
NUM_CORES = 2
NUM_SUBCORES = 16
NUM_WORKERS = NUM_CORES * NUM_SUBCORES
B_PER_W = BATCH // NUM_WORKERS
CHUNK = 128
NCHUNK = B_PER_W // CHUNK
BLK_PER_CHUNK = CHUNK // LANES


def _tr_body(u0, u1, u2, u3, i0, i1, i2, i3, ou_ref, oi_ref):
    eye = (lax.broadcasted_iota(jnp.int32, (128, 128), 0)
           == lax.broadcasted_iota(jnp.int32, (128, 128), 1)
           ).astype(jnp.float32)
    ucat = jnp.concatenate([u0[...], u1[...], u2[...], u3[...]], axis=0)
    ou_ref[...] = lax.dot_general(ucat, eye, (((0,), (0,)), ((), ())),
                                  preferred_element_type=jnp.float32)
    icat = jnp.concatenate([i0[...], i1[...], i2[...], i3[...]], axis=0)
    oi_ref[...] = lax.dot_general(icat, eye, (((0,), (0,)), ((), ())),
                                  preferred_element_type=jnp.float32)


_in_specs = [
    pl.BlockSpec(
        (EMBED, TRW),
        functools.partial(
            lambda i, s: (0, jnp.minimum(s * NBLK + i, -(-NROWS // TRW) - 1)),
            s=s))
    for s in range(GROUP)
]

_tr = pl.pallas_call(
    _tr_body,
    out_shape=(jax.ShapeDtypeStruct((SUB, 128), jnp.float32),
               jax.ShapeDtypeStruct((SUB, 128), jnp.float32)),
    grid=(NBLK,),
    in_specs=_in_specs + _in_specs,
    out_specs=(pl.BlockSpec((TRW, 128), lambda i: (i, 0)),
               pl.BlockSpec((TRW, 128), lambda i: (i, 0))),
)


def _tt_body(uid_hbm, mid_hbm, utab_hbm, itab_hbm, out_hbm,
             uidx_v, midx_v, ucol_v, mcol_v,
             ubuf0, ubuf1, ibuf0, ibuf1, out_v, sem0, sem1):
    wid = lax.axis_index("s") * NUM_CORES + lax.axis_index("c")
    base = wid * B_PER_W

    for j in range(NCHUNK):
        pltpu.sync_copy(uid_hbm.at[pl.ds(base + j * CHUNK, CHUNK)], uidx_v.at[j])
        pltpu.sync_copy(mid_hbm.at[pl.ds(base + j * CHUNK, CHUNK)], midx_v.at[j])

    for j in range(NCHUNK):
        for c in range(CHUNK // LANES):
            sl = pl.ds(c * LANES, LANES)
            gsl = pl.ds(j * CHUNK + c * LANES, LANES)
            for v_ref, col_ref in ((uidx_v, ucol_v), (midx_v, mcol_v)):
                r = v_ref[j, sl]
                s = ((r >= SUB).astype(jnp.int32)
                     + (r >= 2 * SUB).astype(jnp.int32)
                     + (r >= 3 * SUB).astype(jnp.int32))
                col_ref[gsl] = s * EMBED
                v_ref[j, sl] = r - s * SUB

    ubufs = (ubuf0, ubuf1)
    ibufs = (ibuf0, ibuf1)
    sems = (sem0, sem1)

    def fire(k):
        s = sems[k % 2]
        return (pltpu.async_copy(utab_hbm.at[uidx_v.at[k]], ubufs[k % 2], s),
                pltpu.async_copy(itab_hbm.at[midx_v.at[k]], ibufs[k % 2], s))

    lanes = lax.iota(jnp.int32, LANES)
    pend = fire(0)
    for k in range(NCHUNK):
        for h in pend:
            h.wait()
        if k + 1 < NCHUNK:
            pend = fire(k + 1)
        ub = ubufs[k % 2]
        ib = ibufs[k % 2]

        def blk_body(b, carry, ub=ub, ib=ib, k=k):
            rows16 = b * LANES + lanes
            g0 = k * CHUNK
            ucol16 = ucol_v[pl.ds(g0 + b * LANES, LANES)]
            mcol16 = mcol_v[pl.ds(g0 + b * LANES, LANES)]
            acc = jnp.zeros((LANES,), jnp.float32)
            for d in range(EMBED):
                u_d = plsc.load_gather(ub, [rows16, ucol16 + d])
                i_d = plsc.load_gather(ib, [rows16, mcol16 + d])
                acc = acc + u_d * i_d
            out_v[pl.ds(g0 + b * LANES, LANES)] = 1.0 / (1.0 + jnp.exp(-acc))
            return carry

        lax.fori_loop(0, BLK_PER_CHUNK, blk_body, 0)

    pltpu.sync_copy(out_v, out_hbm.at[pl.ds(base, B_PER_W)])


_tt = functools.partial(
    pl.kernel,
    out_type=jax.ShapeDtypeStruct((BATCH,), jnp.float32),
    mesh=plsc.VectorSubcoreMesh(core_axis_name="c", subcore_axis_name="s"),
    scratch_types=[
        pltpu.VMEM((NCHUNK, CHUNK), jnp.int32),
        pltpu.VMEM((NCHUNK, CHUNK), jnp.int32),
        pltpu.VMEM((B_PER_W,), jnp.int32),
        pltpu.VMEM((B_PER_W,), jnp.int32),
        pltpu.VMEM((CHUNK, 128), jnp.float32),
        pltpu.VMEM((CHUNK, 128), jnp.float32),
        pltpu.VMEM((CHUNK, 128), jnp.float32),
        pltpu.VMEM((CHUNK, 128), jnp.float32),
        pltpu.VMEM((B_PER_W,), jnp.float32),
        pltpu.SemaphoreType.DMA,
        pltpu.SemaphoreType.DMA,
    ],
    compiler_params=pltpu.CompilerParams(
        needs_layout_passes=False, use_tc_tiling_on_sc=True),
)(_tt_body)


def kernel(user_id, movie_id, user_table, item_table):
    utt = user_table.T
    itt = item_table.T
    ut4, it4 = _tr(utt, utt, utt, utt, itt, itt, itt, itt)
    return _tt(user_id.astype(jnp.int32), movie_id.astype(jnp.int32),
               ut4, it4)


# speedup vs baseline: 4.5853x; 1.0164x over previous
import functools

import jax
import jax.numpy as jnp
from jax import lax
from jax.experimental import pallas as pl
from jax.experimental.pallas import tpu as pltpu
from jax.experimental.pallas import tpu_sc as plsc

BATCH = 16384
EMBED = 32
NROWS = 1000000
GROUP = 128 // EMBED
TRW = 12288
NBLK = -(-NROWS // (GROUP * TRW))
SUB = NBLK * TRW
LANES = 16
NUM_CORES = 2
NUM_SUBCORES = 16
NUM_WORKERS = NUM_CORES * NUM_SUBCORES
B_PER_W = BATCH // NUM_WORKERS
CHUNK = 128
NCHUNK = B_PER_W // CHUNK
BLK_PER_CHUNK = CHUNK // LANES


def _tr_body(u0, u1, u2, u3, i0, i1, i2, i3, ou_ref, oi_ref):
    eye = (lax.broadcasted_iota(jnp.int32, (128, 128), 0)
           == lax.broadcasted_iota(jnp.int32, (128, 128), 1)
           ).astype(jnp.float32)
    ucat = jnp.concatenate([u0[...], u1[...], u2[...], u3[...]], axis=0)
    ou_ref[...] = lax.dot_general(ucat, eye, (((0,), (0,)), ((), ())),
                                  preferred_element_type=jnp.float32)
    icat = jnp.concatenate([i0[...], i1[...], i2[...], i3[...]], axis=0)
    oi_ref[...] = lax.dot_general(icat, eye, (((0,), (0,)), ((), ())),
                                  preferred_element_type=jnp.float32)


_in_specs = [
    pl.BlockSpec(
        (EMBED, TRW),
        functools.partial(
            lambda i, s: (0, jnp.minimum(s * NBLK + i, -(-NROWS // TRW) - 1)),
            s=s))
    for s in range(GROUP)
]

_tr = pl.pallas_call(
    _tr_body,
    out_shape=(jax.ShapeDtypeStruct((SUB, 128), jnp.float32),
               jax.ShapeDtypeStruct((SUB, 128), jnp.float32)),
    grid=(NBLK,),
    in_specs=_in_specs + _in_specs,
    out_specs=(pl.BlockSpec((TRW, 128), lambda i: (i, 0)),
               pl.BlockSpec((TRW, 128), lambda i: (i, 0))),
)


def _tt_body(uid_hbm, mid_hbm, utab_hbm, itab_hbm, out_hbm,
             uidx_v, midx_v, ucol_v, mcol_v,
             ubuf0, ubuf1, ibuf0, ibuf1, out_v, sem0, sem1):
    wid = lax.axis_index("s") * NUM_CORES + lax.axis_index("c")
    base = wid * B_PER_W

    stage = []
    for j in range(NCHUNK):
        stage.append(pltpu.async_copy(
            uid_hbm.at[pl.ds(base + j * CHUNK, CHUNK)], uidx_v.at[j], sem0))
        stage.append(pltpu.async_copy(
            mid_hbm.at[pl.ds(base + j * CHUNK, CHUNK)], midx_v.at[j], sem0))
    for h in stage:
        h.wait()

    for j in range(NCHUNK):
        for c in range(CHUNK // LANES):
            sl = pl.ds(c * LANES, LANES)
            gsl = pl.ds(j * CHUNK + c * LANES, LANES)
            for v_ref, col_ref in ((uidx_v, ucol_v), (midx_v, mcol_v)):
                r = v_ref[j, sl]
                s = ((r >= SUB).astype(jnp.int32)
                     + (r >= 2 * SUB).astype(jnp.int32)
                     + (r >= 3 * SUB).astype(jnp.int32))
                col_ref[gsl] = s * EMBED
                v_ref[j, sl] = r - s * SUB

    ubufs = (ubuf0, ubuf1)
    ibufs = (ibuf0, ibuf1)
    sems = (sem0, sem1)

    def fire(k):
        s = sems[k % 2]
        return (pltpu.async_copy(utab_hbm.at[uidx_v.at[k]], ubufs[k % 2], s),
                pltpu.async_copy(itab_hbm.at[midx_v.at[k]], ibufs[k % 2], s))

    lanes = lax.iota(jnp.int32, LANES)
    pend = fire(0)
    for k in range(NCHUNK):
        for h in pend:
            h.wait()
        if k + 1 < NCHUNK:
            pend = fire(k + 1)
        ub = ubufs[k % 2]
        ib = ibufs[k % 2]

        def blk_body(b, carry, ub=ub, ib=ib, k=k):
            rows16 = b * LANES + lanes
            g0 = k * CHUNK
            ucol16 = ucol_v[pl.ds(g0 + b * LANES, LANES)]
            mcol16 = mcol_v[pl.ds(g0 + b * LANES, LANES)]
            acc = jnp.zeros((LANES,), jnp.float32)
            for d in range(EMBED):
                u_d = plsc.load_gather(ub, [rows16, ucol16 + d])
                i_d = plsc.load_gather(ib, [rows16, mcol16 + d])
                acc = acc + u_d * i_d
            out_v[pl.ds(g0 + b * LANES, LANES)] = 1.0 / (1.0 + jnp.exp(-acc))
            return carry

        lax.fori_loop(0, BLK_PER_CHUNK, blk_body, 0)

    pltpu.sync_copy(out_v, out_hbm.at[pl.ds(base, B_PER_W)])


_tt = functools.partial(
    pl.kernel,
    out_type=jax.ShapeDtypeStruct((BATCH,), jnp.float32),
    mesh=plsc.VectorSubcoreMesh(core_axis_name="c", subcore_axis_name="s"),
    scratch_types=[
        pltpu.VMEM((NCHUNK, CHUNK), jnp.int32),
        pltpu.VMEM((NCHUNK, CHUNK), jnp.int32),
        pltpu.VMEM((B_PER_W,), jnp.int32),
        pltpu.VMEM((B_PER_W,), jnp.int32),
        pltpu.VMEM((CHUNK, 128), jnp.float32),
        pltpu.VMEM((CHUNK, 128), jnp.float32),
        pltpu.VMEM((CHUNK, 128), jnp.float32),
        pltpu.VMEM((CHUNK, 128), jnp.float32),
        pltpu.VMEM((B_PER_W,), jnp.float32),
        pltpu.SemaphoreType.DMA,
        pltpu.SemaphoreType.DMA,
    ],
    compiler_params=pltpu.CompilerParams(
        needs_layout_passes=False, use_tc_tiling_on_sc=True),
)(_tt_body)


def kernel(user_id, movie_id, user_table, item_table):
    utt = user_table.T
    itt = item_table.T
    ut4, it4 = _tr(utt, utt, utt, utt, itt, itt, itt, itt)
    return _tt(user_id.astype(jnp.int32), movie_id.astype(jnp.int32),
               ut4, it4)
